# baseline (speedup 1.0000x reference)
import jax
import jax.numpy as jnp
from jax.experimental import pallas as pl
from jax.experimental.pallas import tpu as pltpu


def _mask_kernel(idx_ref, out_ref):
    out_ref[...] = jnp.zeros_like(out_ref)

    def body(k, carry):
        r = idx_ref[k]
        out_ref[pl.ds(r, 1), :] = jnp.ones((1, out_ref.shape[1]), dtype=out_ref.dtype)
        return carry

    jax.lax.fori_loop(0, idx_ref.shape[0], body, 0)


def kernel(pos, idx):
    atm, dim = pos.shape[1], pos.shape[2]
    msk = pl.pallas_call(
        _mask_kernel,
        grid_spec=pltpu.PrefetchScalarGridSpec(
            num_scalar_prefetch=1,
            grid=(1,),
            in_specs=[],
            out_specs=pl.BlockSpec((atm, dim), lambda i, idx_ref: (0, 0)),
        ),
        out_shape=jax.ShapeDtypeStruct((atm, dim), jnp.bool_),
    )(idx)
    return (pos, msk[None])
